# Initial kernel scaffold; baseline (speedup 1.0000x reference)
import jax
import jax.numpy as jnp
from jax.experimental import pallas as pl

K = 15
QDIM = 8192


def _prelu(x, a):
    return jnp.where(x >= 0, x, a * x)


def kernel(x, W1, b1, a1, W2, b2, Wc, Wd1, bd1, a2, Wd2, bd2):
    h1 = _prelu(x @ W1 + b1, a1)
    logits = jnp.dot(h1, W2, precision=jax.lax.Precision.HIGHEST) + b2
    _, idx = jax.lax.top_k(logits, K)
    hard = jnp.zeros_like(logits).at[jnp.arange(logits.shape[0])[:, None], idx].set(1.0)
    khot = logits + jax.lax.stop_gradient(hard - logits)
    vq_loss = jnp.float32(0.0)
    k_val = jax.lax.stop_gradient(
        jnp.clip(jnp.sum(khot, axis=-1, keepdims=True), 1.0, float(QDIM))
    )
    khn = khot / k_val
    q = khn @ Wc
    h = _prelu(q @ Wd1 + bd1, a2)
    recon = h @ Wd2 + bd2
    return (recon, khot, vq_loss, k_val)


# speedup vs baseline: 2.5289x; 2.5289x over previous
import functools

import jax
import jax.numpy as jnp
from jax.experimental import pallas as pl
from jax.experimental.pallas import tpu as pltpu

K = 15
NEG = float("-inf")


def _fused_kernel(x_ref, w1_ref, b1_ref, a1_ref, w2_ref, b2_ref, wc_ref,
                  wd1_ref, bd1_ref, a2_ref, wd2_ref, bd2_ref,
                  khot_ref, kval_ref, recon_ref, vbuf):
    xb = x_ref[...].astype(jnp.bfloat16)
    h1 = jnp.dot(xb, w1_ref[...], preferred_element_type=jnp.float32)
    h1 = h1 + b1_ref[...]
    h1 = jnp.where(h1 >= 0, h1, a1_ref[...] * h1).astype(jnp.bfloat16)
    logits = jnp.dot(h1, w2_ref[...], preferred_element_type=jnp.float32)
    vbuf[...] = logits + b2_ref[...]
    khot_ref[...] = jnp.zeros_like(logits)

    def body(_, carry):
        v = vbuf[...]
        m = jnp.max(v, axis=1, keepdims=True)
        sel = v == m
        khot_ref[...] += sel.astype(jnp.float32)
        vbuf[...] = jnp.where(sel, NEG, v)
        return carry

    jax.lax.fori_loop(0, K, body, 0, unroll=False)

    maskf = khot_ref[...]
    kv = jnp.sum(maskf, axis=1, keepdims=True)
    kval_ref[...] = jnp.clip(kv, 1.0, jnp.float32(maskf.shape[1]))

    khn = (maskf / kv).astype(jnp.bfloat16)
    q = jnp.dot(khn, wc_ref[...], preferred_element_type=jnp.float32)

    h = q.astype(jnp.bfloat16)
    h = jnp.dot(h, wd1_ref[...], preferred_element_type=jnp.float32)
    h = h + bd1_ref[...]
    h = jnp.where(h >= 0, h, a2_ref[...] * h).astype(jnp.bfloat16)
    recon = jnp.dot(h, wd2_ref[...], preferred_element_type=jnp.float32)
    recon_ref[...] = recon + bd2_ref[...]


@functools.partial(jax.jit, static_argnames=("bt",))
def _run(x, W1, b1, a1, W2, b2, Wc, Wd1, bd1, a2, Wd2, bd2, bt=128):
    B, D_IN = x.shape
    HID = W1.shape[1]
    QDIM = W2.shape[1]
    EMB = Wc.shape[1]
    grid = (B // bt,)

    def row_block(shape):
        return pl.BlockSpec(shape, lambda i: (i, 0))

    def whole(shape):
        return pl.BlockSpec(shape, lambda i: (0, 0))

    khot, kval, recon = pl.pallas_call(
        _fused_kernel,
        grid=grid,
        in_specs=[
            row_block((bt, D_IN)),
            whole((D_IN, HID)), whole((1, HID)), whole((1, HID)),
            whole((HID, QDIM)), whole((1, QDIM)),
            whole((QDIM, EMB)),
            whole((EMB, HID)), whole((1, HID)), whole((1, HID)),
            whole((HID, D_IN)), whole((1, D_IN)),
        ],
        out_specs=[
            row_block((bt, QDIM)),
            row_block((bt, 1)),
            row_block((bt, D_IN)),
        ],
        out_shape=[
            jax.ShapeDtypeStruct((B, QDIM), jnp.float32),
            jax.ShapeDtypeStruct((B, 1), jnp.float32),
            jax.ShapeDtypeStruct((B, D_IN), jnp.float32),
        ],
        scratch_shapes=[pltpu.VMEM((bt, QDIM), jnp.float32)],
        compiler_params=pltpu.CompilerParams(
            dimension_semantics=("arbitrary",),
        ),
    )(x,
      W1.astype(jnp.bfloat16), b1.reshape(1, -1), a1.reshape(1, -1),
      W2.astype(jnp.bfloat16), b2.reshape(1, -1),
      Wc.astype(jnp.bfloat16),
      Wd1.astype(jnp.bfloat16), bd1.reshape(1, -1), a2.reshape(1, -1),
      Wd2.astype(jnp.bfloat16), bd2.reshape(1, -1))
    return khot, kval, recon


def kernel(x, W1, b1, a1, W2, b2, Wc, Wd1, bd1, a2, Wd2, bd2):
    khot, kval, recon = _run(x, W1, b1, a1, W2, b2, Wc, Wd1, bd1, a2, Wd2, bd2)
    vq_loss = jnp.float32(0.0)
    return (recon, khot, vq_loss, kval)


# speedup vs baseline: 6.8884x; 2.7239x over previous
import functools

import jax
import jax.numpy as jnp
from jax.experimental import pallas as pl
from jax.experimental.pallas import tpu as pltpu

K = 15
NEG = float("-inf")


def _fused_kernel(x_ref, w1_ref, b1_ref, a1_ref, w2_ref, b2_ref, wc_ref,
                  wd1_ref, bd1_ref, a2_ref, wd2_ref, bd2_ref,
                  khot_ref, kval_ref, recon_ref, mbuf):
    xb = x_ref[...].astype(jnp.bfloat16)
    h1 = jnp.dot(xb, w1_ref[...], preferred_element_type=jnp.float32)
    h1 = h1 + b1_ref[...]
    h1 = jnp.where(h1 >= 0, h1, a1_ref[...] * h1).astype(jnp.bfloat16)
    logits = jnp.dot(h1, w2_ref[...], preferred_element_type=jnp.float32)
    L = logits + b2_ref[...]
    qdim = L.shape[1]

    nch = qdim // 8
    M = L[:, :nch]
    for i in range(1, 8):
        M = jnp.maximum(M, L[:, i * nch:(i + 1) * nch])
    mbuf[...] = M

    def mbody(_, t):
        Mc = mbuf[...]
        mx = jnp.max(Mc, axis=1, keepdims=True)
        mbuf[...] = jnp.where(Mc == mx, NEG, Mc)
        return mx

    t = jax.lax.fori_loop(0, K, mbody, jnp.zeros((L.shape[0], 1), jnp.float32),
                          unroll=False)

    maskf = (L >= t).astype(jnp.float32)
    khot_ref[...] = maskf
    cnt = jnp.sum(maskf, axis=1, keepdims=True)

    def fcond(cnt_c):
        return jnp.any(cnt_c > jnp.float32(K))

    def fbody(cnt_c):
        mask_c = khot_ref[...]
        vm = jnp.where(mask_c > 0, L, jnp.float32(jnp.inf))
        mn = jnp.min(vm, axis=1, keepdims=True)
        over = cnt_c > jnp.float32(K)
        rm = jnp.logical_and(vm == mn, over)
        khot_ref[...] = jnp.where(rm, 0.0, mask_c)
        cnt_c = cnt_c - jnp.sum(rm.astype(jnp.float32), axis=1, keepdims=True)
        return cnt_c

    cnt = jax.lax.while_loop(fcond, fbody, cnt)

    maskf = khot_ref[...]
    kval_ref[...] = jnp.clip(cnt, 1.0, jnp.float32(qdim))

    khn = (maskf / cnt).astype(jnp.bfloat16)
    q = jnp.dot(khn, wc_ref[...], preferred_element_type=jnp.float32)

    h = q.astype(jnp.bfloat16)
    h = jnp.dot(h, wd1_ref[...], preferred_element_type=jnp.float32)
    h = h + bd1_ref[...]
    h = jnp.where(h >= 0, h, a2_ref[...] * h).astype(jnp.bfloat16)
    recon = jnp.dot(h, wd2_ref[...], preferred_element_type=jnp.float32)
    recon_ref[...] = recon + bd2_ref[...]


@functools.partial(jax.jit, static_argnames=("bt",))
def _run(x, W1, b1, a1, W2, b2, Wc, Wd1, bd1, a2, Wd2, bd2, bt=128):
    B, D_IN = x.shape
    HID = W1.shape[1]
    QDIM = W2.shape[1]
    EMB = Wc.shape[1]
    grid = (B // bt,)

    def row_block(shape):
        return pl.BlockSpec(shape, lambda i: (i, 0))

    def whole(shape):
        return pl.BlockSpec(shape, lambda i: (0, 0))

    khot, kval, recon = pl.pallas_call(
        _fused_kernel,
        grid=grid,
        in_specs=[
            row_block((bt, D_IN)),
            whole((D_IN, HID)), whole((1, HID)), whole((1, HID)),
            whole((HID, QDIM)), whole((1, QDIM)),
            whole((QDIM, EMB)),
            whole((EMB, HID)), whole((1, HID)), whole((1, HID)),
            whole((HID, D_IN)), whole((1, D_IN)),
        ],
        out_specs=[
            row_block((bt, QDIM)),
            row_block((bt, 1)),
            row_block((bt, D_IN)),
        ],
        out_shape=[
            jax.ShapeDtypeStruct((B, QDIM), jnp.float32),
            jax.ShapeDtypeStruct((B, 1), jnp.float32),
            jax.ShapeDtypeStruct((B, D_IN), jnp.float32),
        ],
        scratch_shapes=[pltpu.VMEM((bt, QDIM // 8), jnp.float32)],
        compiler_params=pltpu.CompilerParams(
            dimension_semantics=("arbitrary",),
        ),
    )(x,
      W1.astype(jnp.bfloat16), b1.reshape(1, -1), a1.reshape(1, -1),
      W2.astype(jnp.bfloat16), b2.reshape(1, -1),
      Wc.astype(jnp.bfloat16),
      Wd1.astype(jnp.bfloat16), bd1.reshape(1, -1), a2.reshape(1, -1),
      Wd2.astype(jnp.bfloat16), bd2.reshape(1, -1))
    return khot, kval, recon


def kernel(x, W1, b1, a1, W2, b2, Wc, Wd1, bd1, a2, Wd2, bd2):
    khot, kval, recon = _run(x, W1, b1, a1, W2, b2, Wc, Wd1, bd1, a2, Wd2, bd2)
    vq_loss = jnp.float32(0.0)
    return (recon, khot, vq_loss, kval)
